# baseline (speedup 1.0000x reference)
import functools

import jax
import jax.numpy as jnp
from jax import lax
from jax.experimental import pallas as pl
from jax.experimental.pallas import tpu as pltpu
from jax.experimental.pallas import tpu_sc as plsc

_N = 10000
_D = 256
_H = 256
_E = 160000

_NC = 2
_NS = 16
_NB = 100
_BW = 100
_NPH = 2
_NBP = _NB // _NPH
_ROWS_PT = 640
_NPD = 10240
_DEG_SL = _NPD // _NS
_RB = 1000
_GRID = _N // _RB

_mesh = plsc.VectorSubcoreMesh(core_axis_name="c", subcore_axis_name="s")


@functools.partial(
    pl.kernel,
    mesh=_mesh,
    out_type=jax.ShapeDtypeStruct((_NPD,), jnp.float32),
    scratch_types=[
        pltpu.VMEM((_NBP, _BW), jnp.int32),
        pltpu.VMEM((_DEG_SL,), jnp.float32),
        pltpu.VMEM_SHARED((_NPD,), jnp.float32),
    ],
)
def _deg_kernel(dst_hbm, deg_hbm, didx_v, ones_v, deg_shared):
    c = lax.axis_index("c")
    s = lax.axis_index("s")

    @pl.when(c == 0)
    def _():
        def fill(i, carry):
            ones_v[pl.ds(i * 16, 16)] = jnp.full((16,), 1.0, jnp.float32)
            return carry
        lax.fori_loop(0, _DEG_SL // 16, fill, 0)
        pltpu.sync_copy(ones_v, deg_shared.at[pl.ds(s * _DEG_SL, _DEG_SL)])
        plsc.subcore_barrier()

        def step(j, carry):
            pltpu.sync_copy(ones_v.at[pl.ds(0, _BW)],
                            deg_shared.at[didx_v.at[j]], add=True)
            return carry

        for p in range(_NPH):
            pltpu.sync_copy(dst_hbm.at[s, p], didx_v)
            lax.fori_loop(0, _NBP, step, 0)
        plsc.subcore_barrier()
        pltpu.sync_copy(deg_shared.at[pl.ds(s * _DEG_SL, _DEG_SL)],
                        deg_hbm.at[pl.ds(s * _DEG_SL, _DEG_SL)])


@functools.partial(
    pl.kernel,
    mesh=_mesh,
    out_type=(jax.ShapeDtypeStruct((_N, 128), jnp.float32),
              jax.ShapeDtypeStruct((_N, 128), jnp.float32)),
    scratch_types=[
        pltpu.VMEM((_NBP, _BW), jnp.int32),
        pltpu.VMEM((_NBP, _BW), jnp.int32),
        pltpu.VMEM((_BW, 128), jnp.float32),
        pltpu.VMEM((_BW, 128), jnp.float32),
        pltpu.SemaphoreType.DMA,
        pltpu.SemaphoreType.DMA,
        pltpu.SemaphoreType.DMA,
        pltpu.VMEM_SHARED((_N, 128), jnp.float32),
    ],
)
def _prop_kernel(ma_hbm, mb_hbm, src_hbm, dst_hbm, outa_hbm, outb_hbm,
                 sidx_v, didx_v, rows0_v, rows1_v, sem0, sem1, semi, acc_sh):
    c = lax.axis_index("c")
    s = lax.axis_index("s")

    def per_tile_rows(do):
        @pl.when(s < _NS - 1)
        def _():
            do(s * _ROWS_PT, _ROWS_PT)

        @pl.when(s == _NS - 1)
        def _():
            do((_NS - 1) * _ROWS_PT, _N - (_NS - 1) * _ROWS_PT)

    def flow(mref, oref):
        def init(b, n):
            pltpu.async_copy(mref.at[pl.ds(b, n)], acc_sh.at[pl.ds(b, n)],
                             semi)
        per_tile_rows(init)
        pltpu.sync_copy(src_hbm.at[s, 0], sidx_v)
        pltpu.sync_copy(dst_hbm.at[s, 0], didx_v)
        pltpu.async_copy(mref.at[sidx_v.at[0]], rows0_v, sem0)
        pltpu.async_copy(mref.at[sidx_v.at[1]], rows1_v, sem1)

        def init_wait(b, n):
            pltpu.make_async_copy(mref.at[pl.ds(b, n)],
                                  acc_sh.at[pl.ds(b, n)], semi).wait()
        per_tile_rows(init_wait)
        plsc.subcore_barrier()

        def step(i, c2):
            j0 = 2 * i
            pltpu.make_async_copy(mref.at[sidx_v.at[j0]], rows0_v,
                                  sem0).wait()
            pltpu.sync_copy(rows0_v, acc_sh.at[didx_v.at[j0]], add=True)

            @pl.when(i < _NBP // 2 - 1)
            def _():
                pltpu.async_copy(mref.at[sidx_v.at[j0 + 2]], rows0_v, sem0)
            pltpu.make_async_copy(mref.at[sidx_v.at[j0 + 1]], rows1_v,
                                  sem1).wait()
            pltpu.sync_copy(rows1_v, acc_sh.at[didx_v.at[j0 + 1]], add=True)

            @pl.when(i < _NBP // 2 - 1)
            def _():
                pltpu.async_copy(mref.at[sidx_v.at[j0 + 3]], rows1_v, sem1)
            return c2

        for p in range(_NPH):
            lax.fori_loop(0, _NBP // 2, step, 0)
            if p + 1 < _NPH:
                pltpu.sync_copy(src_hbm.at[s, p + 1], sidx_v)
                pltpu.sync_copy(dst_hbm.at[s, p + 1], didx_v)
                pltpu.async_copy(mref.at[sidx_v.at[0]], rows0_v, sem0)
                pltpu.async_copy(mref.at[sidx_v.at[1]], rows1_v, sem1)
        plsc.subcore_barrier()

        def writeback(b, n):
            pltpu.sync_copy(acc_sh.at[pl.ds(b, n)], oref.at[pl.ds(b, n)])
        per_tile_rows(writeback)

    @pl.when(c == 0)
    def _():
        flow(ma_hbm, outa_hbm)

    @pl.when(c == 1)
    def _():
        flow(mb_hbm, outb_hbm)


def _k0_body(x_ref, wfc_ref, bfc_ref, w1_ref, deg_ref, r0_ref, ma_ref, mb_ref):
    h = jnp.dot(x_ref[...], wfc_ref[...], preferred_element_type=jnp.float32)
    h = jnp.maximum(h + bfc_ref[...], 0.0)
    r0_ref[...] = h.astype(jnp.bfloat16)
    dinv = lax.rsqrt(deg_ref[...])
    m = jnp.dot(h, w1_ref[...], preferred_element_type=jnp.float32) * dinv
    ma_ref[...] = m[:, :128]
    mb_ref[...] = m[:, 128:]


def _layer_body(acca_ref, accb_ref, deg_ref, b_ref, w_ref, rprev_ref,
                r_ref, ma_ref, mb_ref):
    dinv = lax.rsqrt(deg_ref[...])
    a = jnp.concatenate([acca_ref[...], accb_ref[...]], axis=1)
    h = jnp.maximum(a * dinv + b_ref[...], 0.0)
    r_ref[...] = jnp.maximum(rprev_ref[...].astype(jnp.float32),
                             h).astype(jnp.bfloat16)
    m = jnp.dot(h, w_ref[...], preferred_element_type=jnp.float32) * dinv
    ma_ref[...] = m[:, :128]
    mb_ref[...] = m[:, 128:]


def _final_body(acca_ref, accb_ref, deg_ref, b_ref, rprev_ref, out_ref):
    dinv = lax.rsqrt(deg_ref[...])
    a = jnp.concatenate([acca_ref[...], accb_ref[...]], axis=1)
    h3 = jnp.maximum(a * dinv + b_ref[...], 0.0)
    out_ref[...] = jnp.maximum(rprev_ref[...].astype(jnp.float32), h3)


_row = pl.BlockSpec((_RB, _H), lambda i: (i, 0))
_half = pl.BlockSpec((_RB, 128), lambda i: (i, 0))
_degs = pl.BlockSpec((_RB, 1), lambda i: (i, 0))
_full = pl.BlockSpec((_H, _H), lambda i: (0, 0))
_bias = pl.BlockSpec((1, _H), lambda i: (0, 0))

_k0 = pl.pallas_call(
    _k0_body,
    grid=(_GRID,),
    in_specs=[_row, _full, _bias, _full, _degs],
    out_specs=(_row, _half, _half),
    out_shape=(jax.ShapeDtypeStruct((_N, _H), jnp.bfloat16),
               jax.ShapeDtypeStruct((_N, 128), jnp.float32),
               jax.ShapeDtypeStruct((_N, 128), jnp.float32)),
)

_klayer = pl.pallas_call(
    _layer_body,
    grid=(_GRID,),
    in_specs=[_half, _half, _degs, _bias, _full, _row],
    out_specs=(_row, _half, _half),
    out_shape=(jax.ShapeDtypeStruct((_N, _H), jnp.bfloat16),
               jax.ShapeDtypeStruct((_N, 128), jnp.float32),
               jax.ShapeDtypeStruct((_N, 128), jnp.float32)),
)

_kfinal = pl.pallas_call(
    _final_body,
    grid=(_GRID,),
    in_specs=[_half, _half, _degs, _bias, _row],
    out_specs=_row,
    out_shape=jax.ShapeDtypeStruct((_N, _H), jnp.float32),
)


def kernel(x, edge_index, W_fc, b_fc, W1, b1, W2, b2, W3, b3):
    src = edge_index[0].reshape(_NS, _NPH, _NBP, _BW)
    dst = edge_index[1].reshape(_NS, _NPH, _NBP, _BW)
    deg = _deg_kernel(dst).reshape(_NPD, 1)
    r0, m1a, m1b = _k0(x, W_fc, b_fc.reshape(1, _H), W1, deg)
    acc1a, acc1b = _prop_kernel(m1a, m1b, src, dst)
    r1, m2a, m2b = _klayer(acc1a, acc1b, deg, b1.reshape(1, _H), W2, r0)
    acc2a, acc2b = _prop_kernel(m2a, m2b, src, dst)
    r2, m3a, m3b = _klayer(acc2a, acc2b, deg, b2.reshape(1, _H), W3, r1)
    acc3a, acc3b = _prop_kernel(m3a, m3b, src, dst)
    return _kfinal(acc3a, acc3b, deg, b3.reshape(1, _H), r2)


# speedup vs baseline: 1.0365x; 1.0365x over previous
import functools

import jax
import jax.numpy as jnp
from jax import lax
from jax.experimental import pallas as pl
from jax.experimental.pallas import tpu as pltpu
from jax.experimental.pallas import tpu_sc as plsc

_N = 10000
_D = 256
_H = 256
_E = 160000

_NC = 2
_NS = 16
_NB = 80
_BW = 125
_NPH = 2
_NBP = _NB // _NPH
_ROWS_PT = 640
_NPD = 10240
_DEG_SL = _NPD // _NS
_RB = 1000
_GRID = _N // _RB

_mesh = plsc.VectorSubcoreMesh(core_axis_name="c", subcore_axis_name="s")


@functools.partial(
    pl.kernel,
    mesh=_mesh,
    out_type=jax.ShapeDtypeStruct((_NPD,), jnp.float32),
    scratch_types=[
        pltpu.VMEM((_NBP, _BW), jnp.int32),
        pltpu.VMEM((_DEG_SL,), jnp.float32),
        pltpu.VMEM_SHARED((_NPD,), jnp.float32),
    ],
)
def _deg_kernel(dst_hbm, deg_hbm, didx_v, ones_v, deg_shared):
    c = lax.axis_index("c")
    s = lax.axis_index("s")

    @pl.when(c == 0)
    def _():
        def fill(i, carry):
            ones_v[pl.ds(i * 16, 16)] = jnp.full((16,), 1.0, jnp.float32)
            return carry
        lax.fori_loop(0, _DEG_SL // 16, fill, 0)
        pltpu.sync_copy(ones_v, deg_shared.at[pl.ds(s * _DEG_SL, _DEG_SL)])
        plsc.subcore_barrier()

        def step(j, carry):
            pltpu.sync_copy(ones_v.at[pl.ds(0, _BW)],
                            deg_shared.at[didx_v.at[j]], add=True)
            return carry

        for p in range(_NPH):
            pltpu.sync_copy(dst_hbm.at[s, p], didx_v)
            lax.fori_loop(0, _NBP, step, 0)
        plsc.subcore_barrier()
        pltpu.sync_copy(deg_shared.at[pl.ds(s * _DEG_SL, _DEG_SL)],
                        deg_hbm.at[pl.ds(s * _DEG_SL, _DEG_SL)])


@functools.partial(
    pl.kernel,
    mesh=_mesh,
    out_type=(jax.ShapeDtypeStruct((_N, 128), jnp.float32),
              jax.ShapeDtypeStruct((_N, 128), jnp.float32)),
    scratch_types=[
        pltpu.VMEM((_NBP, _BW), jnp.int32),
        pltpu.VMEM((_NBP, _BW), jnp.int32),
        pltpu.VMEM((_BW, 128), jnp.float32),
        pltpu.VMEM((_BW, 128), jnp.float32),
        pltpu.SemaphoreType.DMA,
        pltpu.SemaphoreType.DMA,
        pltpu.SemaphoreType.DMA,
        pltpu.VMEM_SHARED((_N, 128), jnp.float32),
    ],
)
def _prop_kernel(ma_hbm, mb_hbm, src_hbm, dst_hbm, outa_hbm, outb_hbm,
                 sidx_v, didx_v, rows0_v, rows1_v, sem0, sem1, semi, acc_sh):
    c = lax.axis_index("c")
    s = lax.axis_index("s")

    def per_tile_rows(do):
        @pl.when(s < _NS - 1)
        def _():
            do(s * _ROWS_PT, _ROWS_PT)

        @pl.when(s == _NS - 1)
        def _():
            do((_NS - 1) * _ROWS_PT, _N - (_NS - 1) * _ROWS_PT)

    def flow(mref, oref):
        def init(b, n):
            pltpu.async_copy(mref.at[pl.ds(b, n)], acc_sh.at[pl.ds(b, n)],
                             semi)
        per_tile_rows(init)
        pltpu.sync_copy(src_hbm.at[s, 0], sidx_v)
        pltpu.sync_copy(dst_hbm.at[s, 0], didx_v)
        pltpu.async_copy(mref.at[sidx_v.at[0]], rows0_v, sem0)
        pltpu.async_copy(mref.at[sidx_v.at[1]], rows1_v, sem1)

        def init_wait(b, n):
            pltpu.make_async_copy(mref.at[pl.ds(b, n)],
                                  acc_sh.at[pl.ds(b, n)], semi).wait()
        per_tile_rows(init_wait)
        plsc.subcore_barrier()

        def step(i, c2):
            j0 = 2 * i
            pltpu.make_async_copy(mref.at[sidx_v.at[j0]], rows0_v,
                                  sem0).wait()
            pltpu.sync_copy(rows0_v, acc_sh.at[didx_v.at[j0]], add=True)

            @pl.when(i < _NBP // 2 - 1)
            def _():
                pltpu.async_copy(mref.at[sidx_v.at[j0 + 2]], rows0_v, sem0)
            pltpu.make_async_copy(mref.at[sidx_v.at[j0 + 1]], rows1_v,
                                  sem1).wait()
            pltpu.sync_copy(rows1_v, acc_sh.at[didx_v.at[j0 + 1]], add=True)

            @pl.when(i < _NBP // 2 - 1)
            def _():
                pltpu.async_copy(mref.at[sidx_v.at[j0 + 3]], rows1_v, sem1)
            return c2

        for p in range(_NPH):
            lax.fori_loop(0, _NBP // 2, step, 0)
            if p + 1 < _NPH:
                pltpu.sync_copy(src_hbm.at[s, p + 1], sidx_v)
                pltpu.sync_copy(dst_hbm.at[s, p + 1], didx_v)
                pltpu.async_copy(mref.at[sidx_v.at[0]], rows0_v, sem0)
                pltpu.async_copy(mref.at[sidx_v.at[1]], rows1_v, sem1)
        plsc.subcore_barrier()

        def writeback(b, n):
            pltpu.sync_copy(acc_sh.at[pl.ds(b, n)], oref.at[pl.ds(b, n)])
        per_tile_rows(writeback)

    @pl.when(c == 0)
    def _():
        flow(ma_hbm, outa_hbm)

    @pl.when(c == 1)
    def _():
        flow(mb_hbm, outb_hbm)


def _k0_body(x_ref, wfc_ref, bfc_ref, w1_ref, deg_ref, r0_ref, ma_ref, mb_ref):
    h = jnp.dot(x_ref[...], wfc_ref[...], preferred_element_type=jnp.float32)
    h = jnp.maximum(h + bfc_ref[...], 0.0)
    r0_ref[...] = h.astype(jnp.bfloat16)
    dinv = lax.rsqrt(deg_ref[...])
    m = jnp.dot(h, w1_ref[...], preferred_element_type=jnp.float32) * dinv
    ma_ref[...] = m[:, :128]
    mb_ref[...] = m[:, 128:]


def _layer_body(acca_ref, accb_ref, deg_ref, b_ref, w_ref, rprev_ref,
                r_ref, ma_ref, mb_ref):
    dinv = lax.rsqrt(deg_ref[...])
    a = jnp.concatenate([acca_ref[...], accb_ref[...]], axis=1)
    h = jnp.maximum(a * dinv + b_ref[...], 0.0)
    r_ref[...] = jnp.maximum(rprev_ref[...].astype(jnp.float32),
                             h).astype(jnp.bfloat16)
    m = jnp.dot(h, w_ref[...], preferred_element_type=jnp.float32) * dinv
    ma_ref[...] = m[:, :128]
    mb_ref[...] = m[:, 128:]


def _final_body(acca_ref, accb_ref, deg_ref, b_ref, rprev_ref, out_ref):
    dinv = lax.rsqrt(deg_ref[...])
    a = jnp.concatenate([acca_ref[...], accb_ref[...]], axis=1)
    h3 = jnp.maximum(a * dinv + b_ref[...], 0.0)
    out_ref[...] = jnp.maximum(rprev_ref[...].astype(jnp.float32), h3)


_row = pl.BlockSpec((_RB, _H), lambda i: (i, 0))
_half = pl.BlockSpec((_RB, 128), lambda i: (i, 0))
_degs = pl.BlockSpec((_RB, 1), lambda i: (i, 0))
_full = pl.BlockSpec((_H, _H), lambda i: (0, 0))
_bias = pl.BlockSpec((1, _H), lambda i: (0, 0))

_k0 = pl.pallas_call(
    _k0_body,
    grid=(_GRID,),
    in_specs=[_row, _full, _bias, _full, _degs],
    out_specs=(_row, _half, _half),
    out_shape=(jax.ShapeDtypeStruct((_N, _H), jnp.bfloat16),
               jax.ShapeDtypeStruct((_N, 128), jnp.float32),
               jax.ShapeDtypeStruct((_N, 128), jnp.float32)),
)

_klayer = pl.pallas_call(
    _layer_body,
    grid=(_GRID,),
    in_specs=[_half, _half, _degs, _bias, _full, _row],
    out_specs=(_row, _half, _half),
    out_shape=(jax.ShapeDtypeStruct((_N, _H), jnp.bfloat16),
               jax.ShapeDtypeStruct((_N, 128), jnp.float32),
               jax.ShapeDtypeStruct((_N, 128), jnp.float32)),
)

_kfinal = pl.pallas_call(
    _final_body,
    grid=(_GRID,),
    in_specs=[_half, _half, _degs, _bias, _row],
    out_specs=_row,
    out_shape=jax.ShapeDtypeStruct((_N, _H), jnp.float32),
)


def kernel(x, edge_index, W_fc, b_fc, W1, b1, W2, b2, W3, b3):
    src = edge_index[0].reshape(_NS, _NPH, _NBP, _BW)
    dst = edge_index[1].reshape(_NS, _NPH, _NBP, _BW)
    deg = _deg_kernel(dst).reshape(_NPD, 1)
    r0, m1a, m1b = _k0(x, W_fc, b_fc.reshape(1, _H), W1, deg)
    acc1a, acc1b = _prop_kernel(m1a, m1b, src, dst)
    r1, m2a, m2b = _klayer(acc1a, acc1b, deg, b1.reshape(1, _H), W2, r0)
    acc2a, acc2b = _prop_kernel(m2a, m2b, src, dst)
    r2, m3a, m3b = _klayer(acc2a, acc2b, deg, b2.reshape(1, _H), W3, r1)
    acc3a, acc3b = _prop_kernel(m3a, m3b, src, dst)
    return _kfinal(acc3a, acc3b, deg, b3.reshape(1, _H), r2)


# speedup vs baseline: 1.0491x; 1.0122x over previous
import functools

import jax
import jax.numpy as jnp
from jax import lax
from jax.experimental import pallas as pl
from jax.experimental.pallas import tpu as pltpu
from jax.experimental.pallas import tpu_sc as plsc

_N = 10000
_D = 256
_H = 256
_E = 160000

_NC = 2
_NS = 16
_NB = 80
_BW = 125
_NPH = 2
_NBP = _NB // _NPH
_ROWS_PT = 640
_NPD = 10240
_DEG_SL = _NPD // _NS
_RB = 1000
_GRID = _N // _RB

_mesh = plsc.VectorSubcoreMesh(core_axis_name="c", subcore_axis_name="s")


@functools.partial(
    pl.kernel,
    mesh=_mesh,
    out_type=jax.ShapeDtypeStruct((_NPD,), jnp.float32),
    scratch_types=[
        pltpu.VMEM((_NBP, _BW), jnp.int32),
        pltpu.VMEM((_DEG_SL,), jnp.float32),
        pltpu.VMEM_SHARED((_NPD,), jnp.float32),
    ],
)
def _deg_kernel(edges_hbm, deg_hbm, didx_v, ones_v, deg_shared):
    c = lax.axis_index("c")
    s = lax.axis_index("s")

    @pl.when(c == 0)
    def _():
        def fill(i, carry):
            ones_v[pl.ds(i * 16, 16)] = jnp.full((16,), 1.0, jnp.float32)
            return carry
        lax.fori_loop(0, _DEG_SL // 16, fill, 0)
        pltpu.sync_copy(ones_v, deg_shared.at[pl.ds(s * _DEG_SL, _DEG_SL)])
        plsc.subcore_barrier()

        def step(j, carry):
            pltpu.sync_copy(ones_v.at[pl.ds(0, _BW)],
                            deg_shared.at[didx_v.at[j]], add=True)
            return carry

        for p in range(_NPH):
            pltpu.sync_copy(edges_hbm.at[1, s, p], didx_v)
            lax.fori_loop(0, _NBP, step, 0)
        plsc.subcore_barrier()
        pltpu.sync_copy(deg_shared.at[pl.ds(s * _DEG_SL, _DEG_SL)],
                        deg_hbm.at[pl.ds(s * _DEG_SL, _DEG_SL)])


@functools.partial(
    pl.kernel,
    mesh=_mesh,
    out_type=(jax.ShapeDtypeStruct((_N, 128), jnp.float32),
              jax.ShapeDtypeStruct((_N, 128), jnp.float32)),
    scratch_types=[
        pltpu.VMEM((_NBP, _BW), jnp.int32),
        pltpu.VMEM((_NBP, _BW), jnp.int32),
        pltpu.VMEM((_BW, 128), jnp.float32),
        pltpu.VMEM((_BW, 128), jnp.float32),
        pltpu.SemaphoreType.DMA,
        pltpu.SemaphoreType.DMA,
        pltpu.SemaphoreType.DMA,
        pltpu.VMEM_SHARED((_N, 128), jnp.float32),
    ],
)
def _prop_kernel(ma_hbm, mb_hbm, edges_hbm, outa_hbm, outb_hbm,
                 sidx_v, didx_v, rows0_v, rows1_v, sem0, sem1, semi, acc_sh):
    c = lax.axis_index("c")
    s = lax.axis_index("s")

    def per_tile_rows(do):
        @pl.when(s < _NS - 1)
        def _():
            do(s * _ROWS_PT, _ROWS_PT)

        @pl.when(s == _NS - 1)
        def _():
            do((_NS - 1) * _ROWS_PT, _N - (_NS - 1) * _ROWS_PT)

    def flow(mref, oref):
        def init(b, n):
            pltpu.async_copy(mref.at[pl.ds(b, n)], acc_sh.at[pl.ds(b, n)],
                             semi)
        per_tile_rows(init)
        pltpu.sync_copy(edges_hbm.at[0, s, 0], sidx_v)
        pltpu.sync_copy(edges_hbm.at[1, s, 0], didx_v)
        pltpu.async_copy(mref.at[sidx_v.at[0]], rows0_v, sem0)
        pltpu.async_copy(mref.at[sidx_v.at[1]], rows1_v, sem1)

        def init_wait(b, n):
            pltpu.make_async_copy(mref.at[pl.ds(b, n)],
                                  acc_sh.at[pl.ds(b, n)], semi).wait()
        per_tile_rows(init_wait)
        plsc.subcore_barrier()

        def step(i, c2):
            j0 = 2 * i
            pltpu.make_async_copy(mref.at[sidx_v.at[j0]], rows0_v,
                                  sem0).wait()
            pltpu.sync_copy(rows0_v, acc_sh.at[didx_v.at[j0]], add=True)

            @pl.when(i < _NBP // 2 - 1)
            def _():
                pltpu.async_copy(mref.at[sidx_v.at[j0 + 2]], rows0_v, sem0)
            pltpu.make_async_copy(mref.at[sidx_v.at[j0 + 1]], rows1_v,
                                  sem1).wait()
            pltpu.sync_copy(rows1_v, acc_sh.at[didx_v.at[j0 + 1]], add=True)

            @pl.when(i < _NBP // 2 - 1)
            def _():
                pltpu.async_copy(mref.at[sidx_v.at[j0 + 3]], rows1_v, sem1)
            return c2

        for p in range(_NPH):
            lax.fori_loop(0, _NBP // 2, step, 0)
            if p + 1 < _NPH:
                pltpu.sync_copy(edges_hbm.at[0, s, p + 1], sidx_v)
                pltpu.sync_copy(edges_hbm.at[1, s, p + 1], didx_v)
                pltpu.async_copy(mref.at[sidx_v.at[0]], rows0_v, sem0)
                pltpu.async_copy(mref.at[sidx_v.at[1]], rows1_v, sem1)
        plsc.subcore_barrier()

        def writeback(b, n):
            pltpu.sync_copy(acc_sh.at[pl.ds(b, n)], oref.at[pl.ds(b, n)])
        per_tile_rows(writeback)

    @pl.when(c == 0)
    def _():
        flow(ma_hbm, outa_hbm)

    @pl.when(c == 1)
    def _():
        flow(mb_hbm, outb_hbm)


def _k0_body(x_ref, wfc_ref, bfc_ref, w1_ref, deg_ref, r0_ref, ma_ref, mb_ref):
    h = jnp.dot(x_ref[...], wfc_ref[...], preferred_element_type=jnp.float32)
    h = jnp.maximum(h + bfc_ref[...], 0.0)
    r0_ref[...] = h.astype(jnp.bfloat16)
    dinv = lax.rsqrt(deg_ref[...])
    m = jnp.dot(h, w1_ref[...], preferred_element_type=jnp.float32) * dinv
    ma_ref[...] = m[:, :128]
    mb_ref[...] = m[:, 128:]


def _layer_body(acca_ref, accb_ref, deg_ref, b_ref, w_ref, rprev_ref,
                r_ref, ma_ref, mb_ref):
    dinv = lax.rsqrt(deg_ref[...])
    a = jnp.concatenate([acca_ref[...], accb_ref[...]], axis=1)
    h = jnp.maximum(a * dinv + b_ref[...], 0.0)
    r_ref[...] = jnp.maximum(rprev_ref[...].astype(jnp.float32),
                             h).astype(jnp.bfloat16)
    m = jnp.dot(h, w_ref[...], preferred_element_type=jnp.float32) * dinv
    ma_ref[...] = m[:, :128]
    mb_ref[...] = m[:, 128:]


def _final_body(acca_ref, accb_ref, deg_ref, b_ref, rprev_ref, out_ref):
    dinv = lax.rsqrt(deg_ref[...])
    a = jnp.concatenate([acca_ref[...], accb_ref[...]], axis=1)
    h3 = jnp.maximum(a * dinv + b_ref[...], 0.0)
    out_ref[...] = jnp.maximum(rprev_ref[...].astype(jnp.float32), h3)


_row = pl.BlockSpec((_RB, _H), lambda i: (i, 0))
_half = pl.BlockSpec((_RB, 128), lambda i: (i, 0))
_degs = pl.BlockSpec((_RB, 1), lambda i: (i, 0))
_full = pl.BlockSpec((_H, _H), lambda i: (0, 0))
_bias = pl.BlockSpec((1, _H), lambda i: (0, 0))

_k0 = pl.pallas_call(
    _k0_body,
    grid=(_GRID,),
    in_specs=[_row, _full, _bias, _full, _degs],
    out_specs=(_row, _half, _half),
    out_shape=(jax.ShapeDtypeStruct((_N, _H), jnp.bfloat16),
               jax.ShapeDtypeStruct((_N, 128), jnp.float32),
               jax.ShapeDtypeStruct((_N, 128), jnp.float32)),
)

_klayer = pl.pallas_call(
    _layer_body,
    grid=(_GRID,),
    in_specs=[_half, _half, _degs, _bias, _full, _row],
    out_specs=(_row, _half, _half),
    out_shape=(jax.ShapeDtypeStruct((_N, _H), jnp.bfloat16),
               jax.ShapeDtypeStruct((_N, 128), jnp.float32),
               jax.ShapeDtypeStruct((_N, 128), jnp.float32)),
)

_kfinal = pl.pallas_call(
    _final_body,
    grid=(_GRID,),
    in_specs=[_half, _half, _degs, _bias, _row],
    out_specs=_row,
    out_shape=jax.ShapeDtypeStruct((_N, _H), jnp.float32),
)


def kernel(x, edge_index, W_fc, b_fc, W1, b1, W2, b2, W3, b3):
    edges = edge_index.reshape(2, _NS, _NPH, _NBP, _BW)
    deg = _deg_kernel(edges).reshape(_NPD, 1)
    r0, m1a, m1b = _k0(x, W_fc, b_fc.reshape(1, _H), W1, deg)
    acc1a, acc1b = _prop_kernel(m1a, m1b, edges)
    r1, m2a, m2b = _klayer(acc1a, acc1b, deg, b1.reshape(1, _H), W2, r0)
    acc2a, acc2b = _prop_kernel(m2a, m2b, edges)
    r2, m3a, m3b = _klayer(acc2a, acc2b, deg, b2.reshape(1, _H), W3, r1)
    acc3a, acc3b = _prop_kernel(m3a, m3b, edges)
    return _kfinal(acc3a, acc3b, deg, b3.reshape(1, _H), r2)
